# baseline (device time: 13690 ns/iter reference)
import jax
import jax.numpy as jnp
from jax import lax
from jax.experimental import pallas as pl
from jax.experimental.pallas import tpu as pltpu

N_DEV = 8
E_PER = 2
XOR_MASKS = (1, 3, 4)
MASK_ORDERS = ((1, 3, 4), (3, 4, 1))
N_ROUNDS = 3
N_CHUNKS = 2


def kernel(x, router_W, route_idx, expert_W):
    n, d = x.shape
    h = expert_W.shape[-1]
    h2 = h // N_CHUNKS

    def body(x_ref, rw_ref, idx_ref, w_ref, out_ref,
             send_ref, recv_ref, send_sems, recv_sems):
        my = lax.axis_index("i")

        barrier_sem = pltpu.get_barrier_semaphore()
        for m in XOR_MASKS:
            pl.semaphore_signal(barrier_sem, inc=1, device_id=(my ^ m,),
                                device_id_type=pl.DeviceIdType.MESH)

        rdmas = [[None] * N_CHUNKS for _ in range(N_ROUNDS)]

        def start(r, c, val):
            send_ref[c] = val.astype(jnp.bfloat16)
            rdma = pltpu.make_async_remote_copy(
                src_ref=send_ref.at[c],
                dst_ref=recv_ref.at[r * N_CHUNKS + c],
                send_sem=send_sems.at[r * N_CHUNKS + c],
                recv_sem=recv_sems.at[r * N_CHUNKS + c],
                device_id=(my ^ MASK_ORDERS[c][r],),
                device_id_type=pl.DeviceIdType.MESH,
            )
            rdma.start()
            rdmas[r][c] = rdma

        idx = idx_ref[:, :]
        xv = x_ref[:, :]
        xms = [
            jnp.where(idx == my * E_PER + e, xv, 0.0).astype(jnp.bfloat16)
            for e in range(E_PER)
        ]
        acc_h = []
        for c in range(N_CHUNKS):
            accc = jnp.zeros((n, h2), jnp.float32)
            for e in range(E_PER):
                accc = accc + jnp.dot(
                    xms[e], w_ref[e, :, c * h2:(c + 1) * h2].astype(
                        jnp.bfloat16),
                    preferred_element_type=jnp.float32,
                )
            acc_h.append(accc)
            if c == 0:
                pl.semaphore_wait(barrier_sem, N_ROUNDS)
            start(0, c, accc)

        for r in range(N_ROUNDS):
            for c in range(N_CHUNKS):
                rdmas[r][c].wait()
                acc_h[c] = acc_h[c] + recv_ref[r * N_CHUNKS + c].astype(
                    jnp.float32)
                if r + 1 < N_ROUNDS:
                    start(r + 1, c, acc_h[c])
                else:
                    out_ref[:, c * h2:(c + 1) * h2] = acc_h[c]

    return pl.pallas_call(
        body,
        out_shape=jax.ShapeDtypeStruct((n, h), jnp.float32),
        in_specs=[
            pl.BlockSpec(memory_space=pltpu.VMEM),
            pl.BlockSpec(memory_space=pltpu.VMEM),
            pl.BlockSpec(memory_space=pltpu.VMEM),
            pl.BlockSpec(memory_space=pltpu.VMEM),
        ],
        out_specs=pl.BlockSpec(memory_space=pltpu.VMEM),
        scratch_shapes=[
            pltpu.VMEM((N_CHUNKS, n, h2), jnp.bfloat16),
            pltpu.VMEM((N_ROUNDS * N_CHUNKS, n, h2), jnp.bfloat16),
            pltpu.SemaphoreType.DMA((N_ROUNDS * N_CHUNKS,)),
            pltpu.SemaphoreType.DMA((N_ROUNDS * N_CHUNKS,)),
        ],
        compiler_params=pltpu.CompilerParams(collective_id=0),
    )(x, router_W, route_idx, expert_W)
